# Initial kernel scaffold; baseline (speedup 1.0000x reference)
import jax
import jax.numpy as jnp
from jax.experimental import pallas as pl


def kernel(x, expert_embeddings):
    raise NotImplementedError("write your pallas kernel here")


# speedup vs baseline: 3.3651x; 3.3651x over previous
import functools

import jax
import jax.numpy as jnp
from jax import lax
from jax.experimental import pallas as pl
from jax.experimental.pallas import tpu as pltpu
from jax.experimental.pallas import tpu_sc as plsc

B, S, H, E, K = 4, 8192, 768, 8, 2
T = B * S
NW = 32
TPW = T // NW
L = 16
GROUPS = TPW // L


def _tc_dot_body(x_ref, w_ref, o_ref):
    o_ref[...] = jnp.dot(x_ref[...], w_ref[...],
                         preferred_element_type=jnp.float32)


def _tc_dot(x2d, w_t):
    blk = 2048
    return pl.pallas_call(
        _tc_dot_body,
        grid=(T // blk,),
        in_specs=[
            pl.BlockSpec((blk, H), lambda i: (i, 0)),
            pl.BlockSpec((H, E), lambda i: (0, 0)),
        ],
        out_specs=pl.BlockSpec((blk, E), lambda i: (i, 0)),
        out_shape=jax.ShapeDtypeStruct((T, E), jnp.float32),
    )(x2d, w_t)


def _sc_router_body(dots_hbm, out_hbm, in_v, out_v):
    c = lax.axis_index("c")
    s = lax.axis_index("s")
    wid = s * 2 + c
    base = wid * (TPW * E)
    pltpu.sync_copy(dots_hbm.at[pl.ds(base, TPW * E)], in_v)

    lane = lax.iota(jnp.int32, 16)
    neg_inf = jnp.full((16,), -jnp.inf, jnp.float32)
    one = jnp.full((16,), 1.0, jnp.float32)
    zero = jnp.zeros((16,), jnp.float32)

    def group(g, _):
        flat0 = g * (L * E) + lane * E
        v = [plsc.load_gather(in_v, [flat0 + e]) for e in range(E)]
        m1 = v[0]
        i1 = jnp.zeros((16,), jnp.int32)
        for e in range(1, E):
            gt = v[e] > m1
            m1 = jnp.where(gt, v[e], m1)
            i1 = jnp.where(gt, e, i1)
        m2 = neg_inf
        i2 = jnp.zeros((16,), jnp.int32)
        for e in range(E):
            cand = jnp.where(i1 == e, neg_inf, v[e])
            gt = cand > m2
            m2 = jnp.where(gt, cand, m2)
            i2 = jnp.where(gt, e, i2)
        e2 = jnp.exp(m2 - m1)
        denom = one + e2
        r1 = one / denom
        r2 = e2 / denom
        for e in range(E):
            val = jnp.where(i1 == e, r1, jnp.where(i2 == e, r2, zero))
            plsc.store_scatter(out_v, [flat0 + e], val)
        return 0

    lax.fori_loop(0, GROUPS, group, 0)
    pltpu.sync_copy(out_v, out_hbm.at[pl.ds(base, TPW * E)])


@functools.partial(
    pl.kernel,
    mesh=plsc.VectorSubcoreMesh(core_axis_name="c", subcore_axis_name="s"),
    out_type=jax.ShapeDtypeStruct((T * E,), jnp.float32),
    scratch_types=[
        pltpu.VMEM((TPW * E,), jnp.float32),
        pltpu.VMEM((TPW * E,), jnp.float32),
    ],
    compiler_params=pltpu.CompilerParams(needs_layout_passes=False),
)
def _sc_router(dots_hbm, out_hbm, in_v, out_v):
    _sc_router_body(dots_hbm, out_hbm, in_v, out_v)


def kernel(x, expert_embeddings):
    x2d = x.reshape(T, H)
    w_t = expert_embeddings.T
    dots = _tc_dot(x2d, w_t)
    out_flat = _sc_router(dots.reshape(T * E))
    return out_flat.reshape(B, S, E)


# speedup vs baseline: 3.5401x; 1.0520x over previous
import functools

import jax
import jax.numpy as jnp
from jax import lax
from jax.experimental import pallas as pl
from jax.experimental.pallas import tpu as pltpu
from jax.experimental.pallas import tpu_sc as plsc

B, S, H, E, K = 4, 8192, 768, 8, 2
T = B * S
NW = 32
TPW = T // NW
L = 16
GROUPS = TPW // L

TBLK = 2048


def _tc_dot_body(x_ref, w_ref, o_ref):
    o_ref[...] = jnp.dot(x_ref[...], w_ref[...],
                         preferred_element_type=jnp.float32)


def _tc_dot(x2d, w128):
    return pl.pallas_call(
        _tc_dot_body,
        grid=(T // TBLK,),
        in_specs=[
            pl.BlockSpec((TBLK, H), lambda i: (i, 0)),
            pl.BlockSpec((H, 128), lambda i: (0, 0)),
        ],
        out_specs=pl.BlockSpec((TBLK, 128), lambda i: (i, 0)),
        out_shape=jax.ShapeDtypeStruct((T, 128), jnp.float32),
    )(x2d, w128)


CHUNK = 256


def _sc_router_body(dots_hbm, out_hbm, in_v, out_v):
    c = lax.axis_index("c")
    s = lax.axis_index("s")
    wid = s * 2 + c
    base = wid * TPW

    lane = lax.iota(jnp.int32, 16)
    neg_inf = jnp.full((16,), -jnp.inf, jnp.float32)
    one = jnp.full((16,), 1.0, jnp.float32)
    zero = jnp.zeros((16,), jnp.float32)

    def group(g, _):
        rows = g * L + lane
        ecol = [jnp.full((16,), e, jnp.int32) for e in range(E)]
        v = [plsc.load_gather(in_v, [rows, ecol[e]]) for e in range(E)]
        m1 = v[0]
        i1 = jnp.zeros((16,), jnp.int32)
        for e in range(1, E):
            gt = v[e] > m1
            m1 = jnp.where(gt, v[e], m1)
            i1 = jnp.where(gt, e, i1)
        m2 = neg_inf
        i2 = jnp.zeros((16,), jnp.int32)
        for e in range(E):
            cand = jnp.where(i1 == e, neg_inf, v[e])
            gt = cand > m2
            m2 = jnp.where(gt, cand, m2)
            i2 = jnp.where(gt, e, i2)
        e2 = jnp.exp(m2 - m1)
        denom = one + e2
        r1 = one / denom
        r2 = e2 / denom
        for e in range(E):
            val = jnp.where(i1 == e, r1, jnp.where(i2 == e, r2, zero))
            plsc.store_scatter(out_v, [rows, ecol[e]], val)
        return 0

    for chunk in range(TPW // CHUNK):
        pltpu.sync_copy(dots_hbm.at[pl.ds(base + chunk * CHUNK, CHUNK)], in_v)
        lax.fori_loop(0, CHUNK // L, group, 0)
        pltpu.sync_copy(out_v, out_hbm.at[pl.ds(base + chunk * CHUNK, CHUNK)])


@functools.partial(
    pl.kernel,
    mesh=plsc.VectorSubcoreMesh(core_axis_name="c", subcore_axis_name="s"),
    out_type=jax.ShapeDtypeStruct((T, 128), jnp.float32),
    scratch_types=[
        pltpu.VMEM((CHUNK, 128), jnp.float32),
        pltpu.VMEM((CHUNK, 128), jnp.float32),
    ],
    compiler_params=pltpu.CompilerParams(needs_layout_passes=False),
)
def _sc_router(dots_hbm, out_hbm, in_v, out_v):
    _sc_router_body(dots_hbm, out_hbm, in_v, out_v)


def kernel(x, expert_embeddings):
    x2d = x.reshape(T, H)
    w128 = jnp.zeros((H, 128), jnp.float32).at[:, :E].set(expert_embeddings.T)
    dots128 = _tc_dot(x2d, w128)
    probs128 = _sc_router(dots128)
    return probs128[:, :E].reshape(B, S, E)


# speedup vs baseline: 3.5406x; 1.0001x over previous
import functools

import jax
import jax.numpy as jnp
from jax import lax
from jax.experimental import pallas as pl
from jax.experimental.pallas import tpu as pltpu
from jax.experimental.pallas import tpu_sc as plsc

B, S, H, E, K = 4, 8192, 768, 8, 2
T = B * S
NW = 32
TPW = T // NW
L = 16
GROUPS = TPW // L
CHUNK = 256
NCHK = TPW // CHUNK
GPC = CHUNK // L

TBLK = 2048


def _tc_dot_body(x_ref, w_ref, o_ref):
    o_ref[...] = jnp.dot(x_ref[...], w_ref[...],
                         preferred_element_type=jnp.float32)


def _tc_dot(x2d, w128):
    return pl.pallas_call(
        _tc_dot_body,
        grid=(T // TBLK,),
        in_specs=[
            pl.BlockSpec((TBLK, H), lambda i: (i, 0)),
            pl.BlockSpec((H, 128), lambda i: (0, 0)),
        ],
        out_specs=pl.BlockSpec((TBLK, 128), lambda i: (i, 0)),
        out_shape=jax.ShapeDtypeStruct((T, 128), jnp.float32),
    )(x2d, w128)


def _sc_router_body(dots_hbm, out_hbm, in_a, in_b, out_v, sem_a, sem_b):
    c = lax.axis_index("c")
    s = lax.axis_index("s")
    wid = s * 2 + c
    base = wid * TPW

    lane = lax.iota(jnp.int32, 16)
    neg_inf = jnp.full((16,), -jnp.inf, jnp.float32)
    one = jnp.full((16,), 1.0, jnp.float32)
    zero = jnp.zeros((16,), jnp.float32)
    ecol = [jnp.full((16,), e, jnp.int32) for e in range(E)]

    bufs = [in_a, in_b]
    sems = [sem_a, sem_b]

    def start(ch):
        return pltpu.async_copy(
            dots_hbm.at[pl.ds(base + ch * CHUNK, CHUNK)],
            bufs[ch % 2], sems[ch % 2])

    def make_group(buf, ch):
        def group(g, _):
            rows = g * L + lane
            v = [plsc.load_gather(buf, [rows, ecol[e]]) for e in range(E)]
            m1 = v[0]
            i1 = jnp.zeros((16,), jnp.int32)
            for e in range(1, E):
                gt = v[e] > m1
                m1 = jnp.where(gt, v[e], m1)
                i1 = jnp.where(gt, e, i1)
            m2 = neg_inf
            i2 = jnp.zeros((16,), jnp.int32)
            for e in range(E):
                cand = jnp.where(i1 == e, neg_inf, v[e])
                gt = cand > m2
                m2 = jnp.where(gt, cand, m2)
                i2 = jnp.where(gt, e, i2)
            e2 = jnp.exp(m2 - m1)
            denom = one + e2
            r1 = one / denom
            r2 = e2 / denom
            grow = jnp.full((16,), ch * GPC, jnp.int32) + g
            for e in range(E):
                val = jnp.where(i1 == e, r1, jnp.where(i2 == e, r2, zero))
                plsc.store_scatter(out_v, [grow, lane * E + e], val)
            return 0
        return group

    pending = start(0)
    for ch in range(NCHK):
        nxt = start(ch + 1) if ch + 1 < NCHK else None
        pending.wait()
        lax.fori_loop(0, GPC, make_group(bufs[ch % 2], ch), 0)
        pending = nxt

    pltpu.sync_copy(out_v, out_hbm.at[pl.ds(wid * GROUPS, GROUPS)])


@functools.partial(
    pl.kernel,
    mesh=plsc.VectorSubcoreMesh(core_axis_name="c", subcore_axis_name="s"),
    out_type=jax.ShapeDtypeStruct((T * E // 128, 128), jnp.float32),
    scratch_types=[
        pltpu.VMEM((CHUNK, 128), jnp.float32),
        pltpu.VMEM((CHUNK, 128), jnp.float32),
        pltpu.VMEM((GROUPS, 128), jnp.float32),
        pltpu.SemaphoreType.DMA,
        pltpu.SemaphoreType.DMA,
    ],
    compiler_params=pltpu.CompilerParams(needs_layout_passes=False),
)
def _sc_router(dots_hbm, out_hbm, in_a, in_b, out_v, sem_a, sem_b):
    _sc_router_body(dots_hbm, out_hbm, in_a, in_b, out_v, sem_a, sem_b)


def kernel(x, expert_embeddings):
    x2d = x.reshape(T, H)
    w128 = jnp.zeros((H, 128), jnp.float32).at[:, :E].set(expert_embeddings.T)
    dots128 = _tc_dot(x2d, w128)
    probs_i = _sc_router(dots128)
    return probs_i.reshape(B, S, E)
